# Initial kernel scaffold; baseline (speedup 1.0000x reference)
import jax
import jax.numpy as jnp
from jax.experimental import pallas as pl


def kernel(enc1_w, enc1_asrc, enc1_adst, enc1_b, bn1_g, bn1_b, bn1_m, bn1_v, enc2_w, enc2_asrc, enc2_adst, enc2_b, bn2_g, bn2_b, bn2_m, bn2_v, dec1_w, dec1_b, bn3_g, bn3_b, bn3_m, bn3_v, dec2_w, dec2_b, x, adj, size_factors):
    raise NotImplementedError("write your pallas kernel here")


# speedup vs baseline: 1.3276x; 1.3276x over previous
import functools
import math

import jax
import jax.numpy as jnp
from jax import lax
from jax.experimental import pallas as pl
from jax.experimental.pallas import tpu as pltpu


_NEG = -1e30
_SLOPE = 0.2
_SRC0 = 8
_TM = 512
_TS = 2048


def _proj_kernel(x_ref, w_ref, acat_ref, h_ref, s_ref):
    h32 = jnp.dot(x_ref[...].astype(w_ref.dtype), w_ref[...],
                  preferred_element_type=jnp.float32)
    hb = h32.astype(h_ref.dtype)
    h_ref[...] = hb
    s_ref[...] = jnp.dot(hb, acat_ref[...], preferred_element_type=jnp.float32)


def _attn_accumulate(j, dsc_ref, ssc_ref, h_ref, adj_ref, m_sc, l_sc, acc_sc,
                     *, heads, out_ch, ts):
    @pl.when(j == 0)
    def _init():
        m_sc[...] = jnp.full_like(m_sc, _NEG)
        l_sc[...] = jnp.zeros_like(l_sc)
        acc_sc[...] = jnp.zeros_like(acc_sc)

    madd = (adj_ref[...].astype(jnp.float32) - 1.0) * 1e30
    h_tile = h_ref[pl.ds(j * ts, ts), :]
    for hh in range(heads):
        sl = slice(hh * out_ch, (hh + 1) * out_ch)
        t = dsc_ref[:, hh:hh + 1] + ssc_ref[hh:hh + 1, :]
        e = jnp.maximum(t, _SLOPE * t) + madd
        m_prev = m_sc[:, hh:hh + 1]
        m_new = jnp.maximum(m_prev, jnp.max(e, axis=1, keepdims=True))
        corr = jnp.exp(m_prev - m_new)
        p = jnp.exp(e - m_new)
        l_sc[:, hh:hh + 1] = corr * l_sc[:, hh:hh + 1] + jnp.sum(
            p, axis=1, keepdims=True)
        acc_sc[:, sl] = corr * acc_sc[:, sl] + jnp.dot(
            p.astype(h_tile.dtype), h_tile[:, sl],
            preferred_element_type=jnp.float32)
        m_sc[:, hh:hh + 1] = m_new


def _head_mean(l_sc, acc_sc, scale_ref, shift_ref, *, heads, out_ch):
    total = None
    for hh in range(heads):
        sl = slice(hh * out_ch, (hh + 1) * out_ch)
        inv_l = pl.reciprocal(l_sc[:, hh:hh + 1], approx=True)
        o_h = acc_sc[:, sl] * inv_l
        total = o_h if total is None else total + o_h
    y = total * scale_ref[...] + shift_ref[...]
    return jnp.maximum(y, 0.0)


def _attn1_kernel(dsc_ref, ssc_ref, h_ref, adj_ref, scale_ref, shift_ref,
                  w2_ref, acat2_ref, h2_ref, s2_ref, m_sc, l_sc, acc_sc,
                  *, heads, out_ch, ts):
    j = pl.program_id(1)
    _attn_accumulate(j, dsc_ref, ssc_ref, h_ref, adj_ref, m_sc, l_sc, acc_sc,
                     heads=heads, out_ch=out_ch, ts=ts)

    @pl.when(j == pl.num_programs(1) - 1)
    def _epilogue():
        h1 = _head_mean(l_sc, acc_sc, scale_ref, shift_ref,
                        heads=heads, out_ch=out_ch).astype(w2_ref.dtype)
        h2 = jnp.dot(h1, w2_ref[...], preferred_element_type=jnp.float32)
        h2b = h2.astype(h2_ref.dtype)
        h2_ref[...] = h2b
        s2_ref[...] = jnp.dot(h2b, acat2_ref[...],
                              preferred_element_type=jnp.float32)


def _attn2_kernel(dsc_ref, ssc_ref, h_ref, adj_ref, scale_ref, shift_ref,
                  w1d_ref, shift1d_ref, w2d_ref, b2d_ref, sf_ref, o_ref,
                  m_sc, l_sc, acc_sc, *, heads, out_ch, ts):
    j = pl.program_id(1)
    _attn_accumulate(j, dsc_ref, ssc_ref, h_ref, adj_ref, m_sc, l_sc, acc_sc,
                     heads=heads, out_ch=out_ch, ts=ts)

    @pl.when(j == pl.num_programs(1) - 1)
    def _epilogue():
        z = _head_mean(l_sc, acc_sc, scale_ref, shift_ref,
                       heads=heads, out_ch=out_ch).astype(w1d_ref.dtype)
        y = jnp.dot(z, w1d_ref[...],
                    preferred_element_type=jnp.float32) + shift1d_ref[...]
        y = jnp.maximum(y, 0.0).astype(w2d_ref.dtype)
        o = jnp.dot(y, w2d_ref[...],
                    preferred_element_type=jnp.float32) + b2d_ref[...]
        o_ref[...] = jnp.maximum(o, 0.0) * sf_ref[...]


def _bn_fold(gamma, beta, mean, var, eps=1e-5):
    scale = gamma * lax.rsqrt(var + eps)
    return scale, beta - mean * scale


def _build_att_matrix(asrc, adst):
    heads, c = asrc.shape
    cols = jnp.arange(128)
    rows_head = jnp.arange(heads * c) // c
    dst_flat = adst.reshape(-1)
    src_flat = asrc.reshape(-1)
    a = jnp.where(cols[None, :] == rows_head[:, None], dst_flat[:, None], 0.0)
    a = jnp.where(cols[None, :] == (_SRC0 + rows_head[:, None]),
                  src_flat[:, None], a)
    return a


def _round_up(n, m):
    return ((n + m - 1) // m) * m


def kernel(enc1_w, enc1_asrc, enc1_adst, enc1_b, bn1_g, bn1_b, bn1_m, bn1_v,
           enc2_w, enc2_asrc, enc2_adst, enc2_b, bn2_g, bn2_b, bn2_m, bn2_v,
           dec1_w, dec1_b, bn3_g, bn3_b, bn3_m, bn3_v, dec2_w, dec2_b,
           x, adj, size_factors):
    cdt = jnp.bfloat16
    heads, h_dim = enc1_asrc.shape
    z_dim = enc2_asrc.shape[1]
    n_orig = x.shape[0]
    din = x.shape[1]

    tm, ts = _TM, _TS
    n128 = _round_up(n_orig, 128)
    tm = max(128, min(tm, n128))
    ts = max(128, min(ts, n128))
    lcm = tm * ts // math.gcd(tm, ts)
    n = _round_up(n_orig, lcm)
    if n > n_orig:
        pad = n - n_orig
        x = jnp.pad(x, ((0, pad), (0, 0)))
        adj = jnp.pad(adj, ((0, pad), (0, pad)))
        size_factors = jnp.pad(size_factors, ((0, pad), (0, 0)))

    s1, t1 = _bn_fold(bn1_g, bn1_b, bn1_m, bn1_v)
    s2, t2 = _bn_fold(bn2_g, bn2_b, bn2_m, bn2_v)
    s3, t3 = _bn_fold(bn3_g, bn3_b, bn3_m, bn3_v)

    acat1 = _build_att_matrix(enc1_asrc, enc1_adst).astype(cdt)
    acat2 = _build_att_matrix(enc2_asrc, enc2_adst).astype(cdt)
    w1 = enc1_w.astype(cdt)
    w2 = enc2_w.astype(cdt)

    scale1 = s1 / heads
    shift1 = enc1_b * s1 + t1
    scale2 = s2 / heads
    shift2 = enc2_b * s2 + t2
    w1d = (dec1_w * s3).astype(cdt)
    shift1d = dec1_b * s3 + t3
    w2d = dec2_w.astype(cdt)

    hc1 = heads * h_dim
    hc2 = heads * z_dim
    dout = dec2_w.shape[1]

    hmat1, s1_all = pl.pallas_call(
        _proj_kernel,
        out_shape=(jax.ShapeDtypeStruct((n, hc1), cdt),
                   jax.ShapeDtypeStruct((n, 128), jnp.float32)),
        grid=(n // tm,),
        in_specs=[pl.BlockSpec((tm, din), lambda i: (i, 0)),
                  pl.BlockSpec((din, hc1), lambda i: (0, 0)),
                  pl.BlockSpec((hc1, 128), lambda i: (0, 0))],
        out_specs=(pl.BlockSpec((tm, hc1), lambda i: (i, 0)),
                   pl.BlockSpec((tm, 128), lambda i: (i, 0))),
        compiler_params=pltpu.CompilerParams(
            dimension_semantics=("parallel",)),
    )(x, w1, acat1)

    ssc1 = s1_all[:, _SRC0:_SRC0 + 8].T

    attn1 = functools.partial(_attn1_kernel, heads=heads, out_ch=h_dim, ts=ts)
    hmat2, s2_all = pl.pallas_call(
        attn1,
        out_shape=(jax.ShapeDtypeStruct((n, hc2), cdt),
                   jax.ShapeDtypeStruct((n, 128), jnp.float32)),
        grid=(n // tm, n // ts),
        in_specs=[
            pl.BlockSpec((tm, 128), lambda i, j: (i, 0)),
            pl.BlockSpec((8, ts), lambda i, j: (0, j)),
            pl.BlockSpec((n, hc1), lambda i, j: (0, 0)),
            pl.BlockSpec((tm, ts), lambda i, j: (i, j)),
            pl.BlockSpec((1, h_dim), lambda i, j: (0, 0)),
            pl.BlockSpec((1, h_dim), lambda i, j: (0, 0)),
            pl.BlockSpec((h_dim, hc2), lambda i, j: (0, 0)),
            pl.BlockSpec((hc2, 128), lambda i, j: (0, 0)),
        ],
        out_specs=(pl.BlockSpec((tm, hc2), lambda i, j: (i, 0)),
                   pl.BlockSpec((tm, 128), lambda i, j: (i, 0))),
        scratch_shapes=[pltpu.VMEM((tm, heads), jnp.float32),
                        pltpu.VMEM((tm, heads), jnp.float32),
                        pltpu.VMEM((tm, hc1), jnp.float32)],
        compiler_params=pltpu.CompilerParams(
            dimension_semantics=("parallel", "arbitrary")),
    )(s1_all, ssc1, hmat1, adj, scale1, shift1, w2, acat2)

    ssc2 = s2_all[:, _SRC0:_SRC0 + 8].T

    attn2 = functools.partial(_attn2_kernel, heads=heads, out_ch=z_dim, ts=ts)
    out = pl.pallas_call(
        attn2,
        out_shape=jax.ShapeDtypeStruct((n, dout), jnp.float32),
        grid=(n // tm, n // ts),
        in_specs=[
            pl.BlockSpec((tm, 128), lambda i, j: (i, 0)),
            pl.BlockSpec((8, ts), lambda i, j: (0, j)),
            pl.BlockSpec((n, hc2), lambda i, j: (0, 0)),
            pl.BlockSpec((tm, ts), lambda i, j: (i, j)),
            pl.BlockSpec((1, z_dim), lambda i, j: (0, 0)),
            pl.BlockSpec((1, z_dim), lambda i, j: (0, 0)),
            pl.BlockSpec((z_dim, h_dim), lambda i, j: (0, 0)),
            pl.BlockSpec((1, h_dim), lambda i, j: (0, 0)),
            pl.BlockSpec((h_dim, dout), lambda i, j: (0, 0)),
            pl.BlockSpec((1, dout), lambda i, j: (0, 0)),
            pl.BlockSpec((tm, 1), lambda i, j: (i, 0)),
        ],
        out_specs=pl.BlockSpec((tm, dout), lambda i, j: (i, 0)),
        scratch_shapes=[pltpu.VMEM((tm, heads), jnp.float32),
                        pltpu.VMEM((tm, heads), jnp.float32),
                        pltpu.VMEM((tm, hc2), jnp.float32)],
        compiler_params=pltpu.CompilerParams(
            dimension_semantics=("parallel", "arbitrary")),
    )(s2_all, ssc2, hmat2, adj, scale2, shift2, w1d, shift1d, w2d, dec2_b,
      size_factors)

    return out[:n_orig]


# speedup vs baseline: 1.5740x; 1.1856x over previous
import functools
import math

import jax
import jax.numpy as jnp
from jax import lax
from jax.experimental import pallas as pl
from jax.experimental.pallas import tpu as pltpu


_NEG = -1e30
_SLOPE = 0.2
_SRC0 = 8
_TM = 512
_TS = 4096
_HB = 256


def _proj_kernel(x_ref, w_ref, acat_ref, h_ref, s_ref, *, heads, out_ch):
    h32 = jnp.dot(x_ref[...].astype(w_ref.dtype), w_ref[...],
                  preferred_element_type=jnp.float32)
    hb = h32.astype(h_ref.dtype)
    tm = hb.shape[0]
    one = jnp.ones((tm, 1), h_ref.dtype)
    zero = jnp.zeros((tm, _HB - out_ch - 1), h_ref.dtype)
    blocks = []
    for hh in range(heads):
        blocks += [hb[:, hh * out_ch:(hh + 1) * out_ch], one, zero]
    h_ref[...] = jnp.concatenate(blocks, axis=1)
    s_ref[...] = jnp.dot(hb, acat_ref[...], preferred_element_type=jnp.float32)


def _attn_accumulate(j, dsc_ref, ssc_ref, h_ref, adj_ref, m_sc, acc_sc,
                     *, heads, ts):
    @pl.when(j == 0)
    def _init():
        m_sc[...] = jnp.full_like(m_sc, _NEG)
        acc_sc[...] = jnp.zeros_like(acc_sc)

    madd = (adj_ref[...].astype(jnp.float32) - 1.0) * 1e30
    h_tile = h_ref[pl.ds(j * ts, ts), :]
    for hh in range(heads):
        sl = slice(hh * _HB, (hh + 1) * _HB)
        t = dsc_ref[:, hh:hh + 1] + ssc_ref[hh:hh + 1, :]
        e = jnp.maximum(t, _SLOPE * t) + madd
        m_prev = m_sc[:, hh:hh + 1]
        m_new = jnp.maximum(m_prev, jnp.max(e, axis=1, keepdims=True))
        corr = jnp.exp2(m_prev - m_new)
        p = jnp.exp2(e - m_new)
        acc_sc[:, sl] = corr * acc_sc[:, sl] + jnp.dot(
            p.astype(h_tile.dtype), h_tile[:, sl],
            preferred_element_type=jnp.float32)
        m_sc[:, hh:hh + 1] = m_new


def _head_mean(acc_sc, scale_ref, shift_ref, *, heads, out_ch):
    total = None
    for hh in range(heads):
        sl = slice(hh * _HB, hh * _HB + out_ch)
        lcol = slice(hh * _HB + out_ch, hh * _HB + out_ch + 1)
        inv_l = pl.reciprocal(acc_sc[:, lcol], approx=True)
        o_h = acc_sc[:, sl] * inv_l
        total = o_h if total is None else total + o_h
    y = total * scale_ref[...] + shift_ref[...]
    return jnp.maximum(y, 0.0)


def _attn1_kernel(dsc_ref, ssc_ref, h_ref, adj_ref, scale_ref, shift_ref,
                  w2_ref, acat2_ref, h2_ref, s2_ref, m_sc, acc_sc,
                  *, heads, out_ch, ts):
    j = pl.program_id(1)
    _attn_accumulate(j, dsc_ref, ssc_ref, h_ref, adj_ref, m_sc, acc_sc,
                     heads=heads, ts=ts)

    @pl.when(j == pl.num_programs(1) - 1)
    def _epilogue():
        h1 = _head_mean(acc_sc, scale_ref, shift_ref,
                        heads=heads, out_ch=out_ch).astype(w2_ref.dtype)
        h2 = jnp.dot(h1, w2_ref[...], preferred_element_type=jnp.float32)
        h2b = h2.astype(h2_ref.dtype)
        tm = h2b.shape[0]
        z2 = enc_out = h2b.shape[1] // heads
        one = jnp.ones((tm, 1), h2_ref.dtype)
        zero = jnp.zeros((tm, _HB - z2 - 1), h2_ref.dtype)
        blocks = []
        for hh in range(heads):
            blocks += [h2b[:, hh * z2:(hh + 1) * z2], one, zero]
        h2_ref[...] = jnp.concatenate(blocks, axis=1)
        s2_ref[...] = jnp.dot(h2b, acat2_ref[...],
                              preferred_element_type=jnp.float32)


def _attn2_kernel(dsc_ref, ssc_ref, h_ref, adj_ref, scale_ref, shift_ref,
                  w1d_ref, shift1d_ref, w2d_ref, b2d_ref, sf_ref, o_ref,
                  m_sc, acc_sc, *, heads, out_ch, ts):
    j = pl.program_id(1)
    _attn_accumulate(j, dsc_ref, ssc_ref, h_ref, adj_ref, m_sc, acc_sc,
                     heads=heads, ts=ts)

    @pl.when(j == pl.num_programs(1) - 1)
    def _epilogue():
        z = _head_mean(acc_sc, scale_ref, shift_ref,
                       heads=heads, out_ch=out_ch).astype(w1d_ref.dtype)
        y = jnp.dot(z, w1d_ref[...],
                    preferred_element_type=jnp.float32) + shift1d_ref[...]
        y = jnp.maximum(y, 0.0).astype(w2d_ref.dtype)
        o = jnp.dot(y, w2d_ref[...],
                    preferred_element_type=jnp.float32) + b2d_ref[...]
        o_ref[...] = jnp.maximum(o, 0.0) * sf_ref[...]


def _bn_fold(gamma, beta, mean, var, eps=1e-5):
    scale = gamma * lax.rsqrt(var + eps)
    return scale, beta - mean * scale


def _build_att_matrix(asrc, adst):
    heads, c = asrc.shape
    log2e = jnp.float32(1.4426950408889634)
    cols = jnp.arange(128)
    rows_head = jnp.arange(heads * c) // c
    dst_flat = adst.reshape(-1) * log2e
    src_flat = asrc.reshape(-1) * log2e
    a = jnp.where(cols[None, :] == rows_head[:, None], dst_flat[:, None], 0.0)
    a = jnp.where(cols[None, :] == (_SRC0 + rows_head[:, None]),
                  src_flat[:, None], a)
    return a


def _round_up(n, m):
    return ((n + m - 1) // m) * m


def kernel(enc1_w, enc1_asrc, enc1_adst, enc1_b, bn1_g, bn1_b, bn1_m, bn1_v,
           enc2_w, enc2_asrc, enc2_adst, enc2_b, bn2_g, bn2_b, bn2_m, bn2_v,
           dec1_w, dec1_b, bn3_g, bn3_b, bn3_m, bn3_v, dec2_w, dec2_b,
           x, adj, size_factors):
    cdt = jnp.bfloat16
    heads, h_dim = enc1_asrc.shape
    z_dim = enc2_asrc.shape[1]
    n_orig = x.shape[0]
    din = x.shape[1]

    tm, ts = _TM, _TS
    n128 = _round_up(n_orig, 128)
    tm = max(128, min(tm, n128))
    ts = max(128, min(ts, n128))
    lcm = tm * ts // math.gcd(tm, ts)
    n = _round_up(n_orig, lcm)
    if n > n_orig:
        pad = n - n_orig
        x = jnp.pad(x, ((0, pad), (0, 0)))
        adj = jnp.pad(adj, ((0, pad), (0, pad)))
        size_factors = jnp.pad(size_factors, ((0, pad), (0, 0)))

    s1, t1 = _bn_fold(bn1_g, bn1_b, bn1_m, bn1_v)
    s2, t2 = _bn_fold(bn2_g, bn2_b, bn2_m, bn2_v)
    s3, t3 = _bn_fold(bn3_g, bn3_b, bn3_m, bn3_v)

    acat1 = _build_att_matrix(enc1_asrc, enc1_adst).astype(cdt)
    acat2 = _build_att_matrix(enc2_asrc, enc2_adst).astype(cdt)
    w1 = enc1_w.astype(cdt)
    w2 = enc2_w.astype(cdt)

    scale1 = s1 / heads
    shift1 = enc1_b * s1 + t1
    scale2 = s2 / heads
    shift2 = enc2_b * s2 + t2
    w1d = (dec1_w * s3).astype(cdt)
    shift1d = dec1_b * s3 + t3
    w2d = dec2_w.astype(cdt)

    hb1 = heads * _HB
    hb2 = heads * _HB
    dout = dec2_w.shape[1]

    proj = functools.partial(_proj_kernel, heads=heads, out_ch=h_dim)
    hmat1, s1_all = pl.pallas_call(
        proj,
        out_shape=(jax.ShapeDtypeStruct((n, hb1), cdt),
                   jax.ShapeDtypeStruct((n, 128), jnp.float32)),
        grid=(n // tm,),
        in_specs=[pl.BlockSpec((tm, din), lambda i: (i, 0)),
                  pl.BlockSpec((din, heads * h_dim), lambda i: (0, 0)),
                  pl.BlockSpec((heads * h_dim, 128), lambda i: (0, 0))],
        out_specs=(pl.BlockSpec((tm, hb1), lambda i: (i, 0)),
                   pl.BlockSpec((tm, 128), lambda i: (i, 0))),
        compiler_params=pltpu.CompilerParams(
            dimension_semantics=("parallel",)),
    )(x, w1, acat1)

    ssc1 = s1_all[:, _SRC0:_SRC0 + 8].T

    attn1 = functools.partial(_attn1_kernel, heads=heads, out_ch=h_dim, ts=ts)
    hmat2, s2_all = pl.pallas_call(
        attn1,
        out_shape=(jax.ShapeDtypeStruct((n, hb2), cdt),
                   jax.ShapeDtypeStruct((n, 128), jnp.float32)),
        grid=(n // tm, n // ts),
        in_specs=[
            pl.BlockSpec((tm, 128), lambda i, j: (i, 0)),
            pl.BlockSpec((8, ts), lambda i, j: (0, j)),
            pl.BlockSpec((n, hb1), lambda i, j: (0, 0)),
            pl.BlockSpec((tm, ts), lambda i, j: (i, j)),
            pl.BlockSpec((1, h_dim), lambda i, j: (0, 0)),
            pl.BlockSpec((1, h_dim), lambda i, j: (0, 0)),
            pl.BlockSpec((h_dim, heads * z_dim), lambda i, j: (0, 0)),
            pl.BlockSpec((heads * z_dim, 128), lambda i, j: (0, 0)),
        ],
        out_specs=(pl.BlockSpec((tm, hb2), lambda i, j: (i, 0)),
                   pl.BlockSpec((tm, 128), lambda i, j: (i, 0))),
        scratch_shapes=[pltpu.VMEM((tm, heads), jnp.float32),
                        pltpu.VMEM((tm, hb1), jnp.float32)],
        compiler_params=pltpu.CompilerParams(
            dimension_semantics=("parallel", "arbitrary")),
    )(s1_all, ssc1, hmat1, adj, scale1, shift1, w2, acat2)

    ssc2 = s2_all[:, _SRC0:_SRC0 + 8].T

    attn2 = functools.partial(_attn2_kernel, heads=heads, out_ch=z_dim, ts=ts)
    out = pl.pallas_call(
        attn2,
        out_shape=jax.ShapeDtypeStruct((n, dout), jnp.float32),
        grid=(n // tm, n // ts),
        in_specs=[
            pl.BlockSpec((tm, 128), lambda i, j: (i, 0)),
            pl.BlockSpec((8, ts), lambda i, j: (0, j)),
            pl.BlockSpec((n, hb2), lambda i, j: (0, 0)),
            pl.BlockSpec((tm, ts), lambda i, j: (i, j)),
            pl.BlockSpec((1, z_dim), lambda i, j: (0, 0)),
            pl.BlockSpec((1, z_dim), lambda i, j: (0, 0)),
            pl.BlockSpec((z_dim, h_dim), lambda i, j: (0, 0)),
            pl.BlockSpec((1, h_dim), lambda i, j: (0, 0)),
            pl.BlockSpec((h_dim, dout), lambda i, j: (0, 0)),
            pl.BlockSpec((1, dout), lambda i, j: (0, 0)),
            pl.BlockSpec((tm, 1), lambda i, j: (i, 0)),
        ],
        out_specs=pl.BlockSpec((tm, dout), lambda i, j: (i, 0)),
        scratch_shapes=[pltpu.VMEM((tm, heads), jnp.float32),
                        pltpu.VMEM((tm, hb2), jnp.float32)],
        compiler_params=pltpu.CompilerParams(
            dimension_semantics=("parallel", "arbitrary")),
    )(s2_all, ssc2, hmat2, adj, scale2, shift2, w1d, shift1d, w2d, dec2_b,
      size_factors)

    return out[:n_orig]


# speedup vs baseline: 1.6389x; 1.0412x over previous
import functools
import math

import jax
import jax.numpy as jnp
from jax import lax
from jax.experimental import pallas as pl
from jax.experimental.pallas import tpu as pltpu


_SLOPE = 0.2
_SRC0 = 8
_TM = 256
_HB = 256
_CLO = -80.0
_CHI = 30.0


def _build_rhs(hb, s_all, *, heads, out_ch, dtype):
    tm = hb.shape[0]
    zero = jnp.zeros((tm, _HB - out_ch - 1), dtype)
    b1, b2 = [], []
    for hh in range(heads):
        v = s_all[:, _SRC0 + hh:_SRC0 + hh + 1]
        e1 = jnp.exp2(jnp.clip(v, _CLO, _CHI)).astype(dtype)
        e2 = jnp.exp2(jnp.clip(_SLOPE * v, _CLO, _CHI)).astype(dtype)
        h_h = hb[:, hh * out_ch:(hh + 1) * out_ch]
        b1 += [h_h * e1, e1, zero]
        b2 += [h_h * e2, e2, zero]
    return jnp.concatenate(b1, axis=1), jnp.concatenate(b2, axis=1)


def _proj_kernel(x_ref, w_ref, acat_ref, r1_ref, r2_ref, s_ref,
                 *, heads, out_ch):
    h32 = jnp.dot(x_ref[...].astype(w_ref.dtype), w_ref[...],
                  preferred_element_type=jnp.float32)
    hb = h32.astype(r1_ref.dtype)
    s_all = jnp.dot(hb, acat_ref[...], preferred_element_type=jnp.float32)
    s_ref[...] = s_all
    r1, r2 = _build_rhs(hb, s_all, heads=heads, out_ch=out_ch,
                        dtype=r1_ref.dtype)
    r1_ref[...] = r1
    r2_ref[...] = r2


def _attn_body(dsc_ref, ssc_ref, r1_ref, r2_ref, adj_ref, scale_ref,
               shift_ref, *, heads, out_ch):
    maskb = adj_ref[...].astype(jnp.bfloat16)
    madd = (maskb - 1.0) * jnp.bfloat16(1e30)
    total = None
    for hh in range(heads):
        u = dsc_ref[:, hh:hh + 1]
        ub = u.astype(jnp.bfloat16)
        t = ub + ssc_ref[hh:hh + 1, :]
        r = jnp.max(t + madd, axis=1, keepdims=True)
        m = jnp.maximum(r, _SLOPE * r).astype(jnp.float32)
        m1 = jnp.where(t > 0, maskb, jnp.bfloat16(0.0))
        m2 = maskb - m1
        sl = slice(hh * _HB, (hh + 1) * _HB)
        d1 = jnp.dot(m1, r1_ref[:, sl], preferred_element_type=jnp.float32)
        d2 = jnp.dot(m2, r2_ref[:, sl], preferred_element_type=jnp.float32)
        f1 = jnp.exp2(jnp.clip(u - m, _CLO, _CHI))
        f2 = jnp.exp2(jnp.clip(_SLOPE * u - m, _CLO, _CHI))
        num = f1 * d1[:, :out_ch] + f2 * d2[:, :out_ch]
        den = f1 * d1[:, out_ch:out_ch + 1] + f2 * d2[:, out_ch:out_ch + 1]
        o_h = num * pl.reciprocal(den, approx=True)
        total = o_h if total is None else total + o_h
    y = total * scale_ref[...] + shift_ref[...]
    return jnp.maximum(y, 0.0)


def _attn1_kernel(dsc_ref, ssc_ref, r1_ref, r2_ref, adj_ref, scale_ref,
                  shift_ref, w2_ref, acat2_ref, r1o_ref, r2o_ref, s2_ref,
                  *, heads, out_ch, out_ch2):
    h1 = _attn_body(dsc_ref, ssc_ref, r1_ref, r2_ref, adj_ref, scale_ref,
                    shift_ref, heads=heads, out_ch=out_ch)
    h1b = h1.astype(w2_ref.dtype)
    h2 = jnp.dot(h1b, w2_ref[...], preferred_element_type=jnp.float32)
    h2b = h2.astype(r1o_ref.dtype)
    s2 = jnp.dot(h2b, acat2_ref[...], preferred_element_type=jnp.float32)
    s2_ref[...] = s2
    r1, r2 = _build_rhs(h2b, s2, heads=heads, out_ch=out_ch2,
                        dtype=r1o_ref.dtype)
    r1o_ref[...] = r1
    r2o_ref[...] = r2


def _attn2_kernel(dsc_ref, ssc_ref, r1_ref, r2_ref, adj_ref, scale_ref,
                  shift_ref, w1d_ref, shift1d_ref, w2d_ref, b2d_ref, sf_ref,
                  o_ref, *, heads, out_ch):
    z = _attn_body(dsc_ref, ssc_ref, r1_ref, r2_ref, adj_ref, scale_ref,
                   shift_ref, heads=heads, out_ch=out_ch)
    zb = z.astype(w1d_ref.dtype)
    y = jnp.dot(zb, w1d_ref[...],
                preferred_element_type=jnp.float32) + shift1d_ref[...]
    y = jnp.maximum(y, 0.0).astype(w2d_ref.dtype)
    o = jnp.dot(y, w2d_ref[...],
                preferred_element_type=jnp.float32) + b2d_ref[...]
    o_ref[...] = jnp.maximum(o, 0.0) * sf_ref[...]


def _bn_fold(gamma, beta, mean, var, eps=1e-5):
    scale = gamma * lax.rsqrt(var + eps)
    return scale, beta - mean * scale


def _build_att_matrix(asrc, adst):
    heads, c = asrc.shape
    log2e = jnp.float32(1.4426950408889634)
    cols = jnp.arange(128)
    rows_head = jnp.arange(heads * c) // c
    dst_flat = adst.reshape(-1) * log2e
    src_flat = asrc.reshape(-1) * log2e
    a = jnp.where(cols[None, :] == rows_head[:, None], dst_flat[:, None], 0.0)
    a = jnp.where(cols[None, :] == (_SRC0 + rows_head[:, None]),
                  src_flat[:, None], a)
    return a


def _round_up(n, m):
    return ((n + m - 1) // m) * m


def kernel(enc1_w, enc1_asrc, enc1_adst, enc1_b, bn1_g, bn1_b, bn1_m, bn1_v,
           enc2_w, enc2_asrc, enc2_adst, enc2_b, bn2_g, bn2_b, bn2_m, bn2_v,
           dec1_w, dec1_b, bn3_g, bn3_b, bn3_m, bn3_v, dec2_w, dec2_b,
           x, adj, size_factors):
    cdt = jnp.bfloat16
    heads, h_dim = enc1_asrc.shape
    z_dim = enc2_asrc.shape[1]
    n_orig = x.shape[0]
    din = x.shape[1]

    tm = max(128, min(_TM, _round_up(n_orig, 128)))
    n = _round_up(n_orig, tm)
    if n > n_orig:
        pad = n - n_orig
        x = jnp.pad(x, ((0, pad), (0, 0)))
        adj = jnp.pad(adj, ((0, pad), (0, pad)))
        size_factors = jnp.pad(size_factors, ((0, pad), (0, 0)))

    s1, t1 = _bn_fold(bn1_g, bn1_b, bn1_m, bn1_v)
    s2, t2 = _bn_fold(bn2_g, bn2_b, bn2_m, bn2_v)
    s3, t3 = _bn_fold(bn3_g, bn3_b, bn3_m, bn3_v)

    acat1 = _build_att_matrix(enc1_asrc, enc1_adst).astype(cdt)
    acat2 = _build_att_matrix(enc2_asrc, enc2_adst).astype(cdt)
    w1 = enc1_w.astype(cdt)
    w2 = enc2_w.astype(cdt)

    scale1 = s1 / heads
    shift1 = enc1_b * s1 + t1
    scale2 = s2 / heads
    shift2 = enc2_b * s2 + t2
    w1d = (dec1_w * s3).astype(cdt)
    shift1d = dec1_b * s3 + t3
    w2d = dec2_w.astype(cdt)

    hb1 = heads * _HB
    hb2 = heads * _HB
    dout = dec2_w.shape[1]
    vlim = pltpu.CompilerParams(
        dimension_semantics=("parallel",),
        vmem_limit_bytes=56 * 1024 * 1024)

    proj = functools.partial(_proj_kernel, heads=heads, out_ch=h_dim)
    rhs1a, rhs2a, s1_all = pl.pallas_call(
        proj,
        out_shape=(jax.ShapeDtypeStruct((n, hb1), cdt),
                   jax.ShapeDtypeStruct((n, hb1), cdt),
                   jax.ShapeDtypeStruct((n, 128), jnp.float32)),
        grid=(n // tm,),
        in_specs=[pl.BlockSpec((tm, din), lambda i: (i, 0)),
                  pl.BlockSpec((din, heads * h_dim), lambda i: (0, 0)),
                  pl.BlockSpec((heads * h_dim, 128), lambda i: (0, 0))],
        out_specs=(pl.BlockSpec((tm, hb1), lambda i: (i, 0)),
                   pl.BlockSpec((tm, hb1), lambda i: (i, 0)),
                   pl.BlockSpec((tm, 128), lambda i: (i, 0))),
        compiler_params=vlim,
    )(x, w1, acat1)

    ssc1 = s1_all[:, _SRC0:_SRC0 + 8].T.astype(cdt)

    attn1 = functools.partial(_attn1_kernel, heads=heads, out_ch=h_dim,
                              out_ch2=z_dim)
    rhs1b, rhs2b, s2_all = pl.pallas_call(
        attn1,
        out_shape=(jax.ShapeDtypeStruct((n, hb2), cdt),
                   jax.ShapeDtypeStruct((n, hb2), cdt),
                   jax.ShapeDtypeStruct((n, 128), jnp.float32)),
        grid=(n // tm,),
        in_specs=[
            pl.BlockSpec((tm, 128), lambda i: (i, 0)),
            pl.BlockSpec((8, n), lambda i: (0, 0)),
            pl.BlockSpec((n, hb1), lambda i: (0, 0)),
            pl.BlockSpec((n, hb1), lambda i: (0, 0)),
            pl.BlockSpec((tm, n), lambda i: (i, 0)),
            pl.BlockSpec((1, h_dim), lambda i: (0, 0)),
            pl.BlockSpec((1, h_dim), lambda i: (0, 0)),
            pl.BlockSpec((h_dim, heads * z_dim), lambda i: (0, 0)),
            pl.BlockSpec((heads * z_dim, 128), lambda i: (0, 0)),
        ],
        out_specs=(pl.BlockSpec((tm, hb2), lambda i: (i, 0)),
                   pl.BlockSpec((tm, hb2), lambda i: (i, 0)),
                   pl.BlockSpec((tm, 128), lambda i: (i, 0))),
        compiler_params=vlim,
    )(s1_all, ssc1, rhs1a, rhs2a, adj, scale1, shift1, w2, acat2)

    ssc2 = s2_all[:, _SRC0:_SRC0 + 8].T.astype(cdt)

    attn2 = functools.partial(_attn2_kernel, heads=heads, out_ch=z_dim)
    out = pl.pallas_call(
        attn2,
        out_shape=jax.ShapeDtypeStruct((n, dout), jnp.float32),
        grid=(n // tm,),
        in_specs=[
            pl.BlockSpec((tm, 128), lambda i: (i, 0)),
            pl.BlockSpec((8, n), lambda i: (0, 0)),
            pl.BlockSpec((n, hb2), lambda i: (0, 0)),
            pl.BlockSpec((n, hb2), lambda i: (0, 0)),
            pl.BlockSpec((tm, n), lambda i: (i, 0)),
            pl.BlockSpec((1, z_dim), lambda i: (0, 0)),
            pl.BlockSpec((1, z_dim), lambda i: (0, 0)),
            pl.BlockSpec((z_dim, h_dim), lambda i: (0, 0)),
            pl.BlockSpec((1, h_dim), lambda i: (0, 0)),
            pl.BlockSpec((h_dim, dout), lambda i: (0, 0)),
            pl.BlockSpec((1, dout), lambda i: (0, 0)),
            pl.BlockSpec((tm, 1), lambda i: (i, 0)),
        ],
        out_specs=pl.BlockSpec((tm, dout), lambda i: (i, 0)),
        compiler_params=vlim,
    )(s2_all, ssc2, rhs1b, rhs2b, adj, scale2, shift2, w1d, shift1d, w2d,
      dec2_b, size_factors)

    return out[:n_orig]


# speedup vs baseline: 3.3044x; 2.0163x over previous
import functools
import math

import jax
import jax.numpy as jnp
from jax import lax
from jax.experimental import pallas as pl
from jax.experimental.pallas import tpu as pltpu


_SLOPE = 0.2
_SRC0 = 8
_TM = 256
_HB = 256
_CLO = -80.0
_CHI = 30.0


def _build_rhs(hb, s_all, *, heads, out_ch, dtype):
    tm = hb.shape[0]
    zero = jnp.zeros((tm, _HB - out_ch - 1), dtype)
    blocks = []
    for hh in range(heads):
        v = s_all[:, _SRC0 + hh:_SRC0 + hh + 1]
        e2 = jnp.exp2(jnp.clip(_SLOPE * v, _CLO, _CHI)).astype(dtype)
        h_h = hb[:, hh * out_ch:(hh + 1) * out_ch]
        blocks += [h_h * e2, e2, zero]
    return jnp.concatenate(blocks, axis=1)


def _proj_kernel(x_ref, w_ref, acat_ref, r2_ref, s_ref, *, heads, out_ch):
    h32 = jnp.dot(x_ref[...].astype(w_ref.dtype), w_ref[...],
                  preferred_element_type=jnp.float32)
    hb = h32.astype(r2_ref.dtype)
    s_all = jnp.dot(hb, acat_ref[...], preferred_element_type=jnp.float32)
    s_ref[...] = s_all
    r2_ref[...] = _build_rhs(hb, s_all, heads=heads, out_ch=out_ch,
                             dtype=r2_ref.dtype)


def _attn_body(dsc_ref, ssc_ref, g_ref, vg_ref, r2_ref, adj_ref, scale_ref,
               shift_ref, *, heads, out_ch):
    maskb = adj_ref[...].astype(jnp.bfloat16)
    total = None
    for hh in range(heads):
        u = dsc_ref[:, hh:hh + 1]
        s = u + vg_ref[0:1, hh:hh + 1]
        m = jnp.maximum(s, _SLOPE * s)
        f1 = jnp.exp2(jnp.clip(u - m, _CLO, _CHI)).astype(jnp.bfloat16)
        f2 = jnp.exp2(jnp.clip(_SLOPE * u - m, _CLO, _CHI)).astype(
            jnp.bfloat16)
        t = u.astype(jnp.bfloat16) + ssc_ref[hh:hh + 1, :]
        a = f1 * g_ref[hh:hh + 1, :]
        p = jnp.where(t > 0, a, f2) * maskb
        sl = slice(hh * _HB, (hh + 1) * _HB)
        d = jnp.dot(p, r2_ref[:, sl], preferred_element_type=jnp.float32)
        o_h = d[:, :out_ch] * pl.reciprocal(d[:, out_ch:out_ch + 1],
                                            approx=True)
        total = o_h if total is None else total + o_h
    y = total * scale_ref[...] + shift_ref[...]
    return jnp.maximum(y, 0.0)


def _attn1_kernel(dsc_ref, ssc_ref, g_ref, vg_ref, r2_ref, adj_ref, scale_ref,
                  shift_ref, w2_ref, acat2_ref, r2o_ref, s2_ref,
                  *, heads, out_ch, out_ch2):
    h1 = _attn_body(dsc_ref, ssc_ref, g_ref, vg_ref, r2_ref, adj_ref,
                    scale_ref, shift_ref, heads=heads, out_ch=out_ch)
    h1b = h1.astype(w2_ref.dtype)
    h2 = jnp.dot(h1b, w2_ref[...], preferred_element_type=jnp.float32)
    h2b = h2.astype(r2o_ref.dtype)
    s2 = jnp.dot(h2b, acat2_ref[...], preferred_element_type=jnp.float32)
    s2_ref[...] = s2
    r2o_ref[...] = _build_rhs(h2b, s2, heads=heads, out_ch=out_ch2,
                              dtype=r2o_ref.dtype)


def _attn2_kernel(dsc_ref, ssc_ref, g_ref, vg_ref, r2_ref, adj_ref, scale_ref,
                  shift_ref, w1d_ref, shift1d_ref, w2d_ref, b2d_ref, sf_ref,
                  o_ref, *, heads, out_ch):
    z = _attn_body(dsc_ref, ssc_ref, g_ref, vg_ref, r2_ref, adj_ref,
                   scale_ref, shift_ref, heads=heads, out_ch=out_ch)
    zb = z.astype(w1d_ref.dtype)
    y = jnp.dot(zb, w1d_ref[...],
                preferred_element_type=jnp.float32) + shift1d_ref[...]
    y = jnp.maximum(y, 0.0).astype(w2d_ref.dtype)
    o = jnp.dot(y, w2d_ref[...],
                preferred_element_type=jnp.float32) + b2d_ref[...]
    o_ref[...] = jnp.maximum(o, 0.0) * sf_ref[...]


def _bn_fold(gamma, beta, mean, var, eps=1e-5):
    scale = gamma * lax.rsqrt(var + eps)
    return scale, beta - mean * scale


def _build_att_matrix(asrc, adst):
    heads, c = asrc.shape
    log2e = jnp.float32(1.4426950408889634)
    cols = jnp.arange(128)
    rows_head = jnp.arange(heads * c) // c
    dst_flat = adst.reshape(-1) * log2e
    src_flat = asrc.reshape(-1) * log2e
    a = jnp.where(cols[None, :] == rows_head[:, None], dst_flat[:, None], 0.0)
    a = jnp.where(cols[None, :] == (_SRC0 + rows_head[:, None]),
                  src_flat[:, None], a)
    return a


def _side_slabs(s_all, heads):
    ssc = s_all[:, _SRC0:_SRC0 + 8].T
    g = jnp.exp2(jnp.clip((1.0 - _SLOPE) * ssc, _CLO, _CHI)).astype(
        jnp.bfloat16)
    vg = jnp.max(ssc[:heads], axis=1)
    vg = jnp.pad(vg, (0, 128 - heads)).reshape(1, 128)
    return ssc.astype(jnp.bfloat16), g, vg


def _round_up(n, m):
    return ((n + m - 1) // m) * m


def kernel(enc1_w, enc1_asrc, enc1_adst, enc1_b, bn1_g, bn1_b, bn1_m, bn1_v,
           enc2_w, enc2_asrc, enc2_adst, enc2_b, bn2_g, bn2_b, bn2_m, bn2_v,
           dec1_w, dec1_b, bn3_g, bn3_b, bn3_m, bn3_v, dec2_w, dec2_b,
           x, adj, size_factors):
    cdt = jnp.bfloat16
    heads, h_dim = enc1_asrc.shape
    z_dim = enc2_asrc.shape[1]
    n_orig = x.shape[0]
    din = x.shape[1]

    tm = max(128, min(_TM, _round_up(n_orig, 128)))
    n = _round_up(n_orig, tm)
    if n > n_orig:
        pad = n - n_orig
        x = jnp.pad(x, ((0, pad), (0, 0)))
        adj = jnp.pad(adj, ((0, pad), (0, pad)))
        size_factors = jnp.pad(size_factors, ((0, pad), (0, 0)))

    s1, t1 = _bn_fold(bn1_g, bn1_b, bn1_m, bn1_v)
    s2, t2 = _bn_fold(bn2_g, bn2_b, bn2_m, bn2_v)
    s3, t3 = _bn_fold(bn3_g, bn3_b, bn3_m, bn3_v)

    acat1 = _build_att_matrix(enc1_asrc, enc1_adst).astype(cdt)
    acat2 = _build_att_matrix(enc2_asrc, enc2_adst).astype(cdt)
    w1 = enc1_w.astype(cdt)
    w2 = enc2_w.astype(cdt)

    scale1 = s1 / heads
    shift1 = enc1_b * s1 + t1
    scale2 = s2 / heads
    shift2 = enc2_b * s2 + t2
    w1d = (dec1_w * s3).astype(cdt)
    shift1d = dec1_b * s3 + t3
    w2d = dec2_w.astype(cdt)

    hb1 = heads * _HB
    hb2 = heads * _HB
    dout = dec2_w.shape[1]
    vlim = pltpu.CompilerParams(
        dimension_semantics=("parallel",),
        vmem_limit_bytes=56 * 1024 * 1024)

    proj = functools.partial(_proj_kernel, heads=heads, out_ch=h_dim)
    rhs_a, s1_all = pl.pallas_call(
        proj,
        out_shape=(jax.ShapeDtypeStruct((n, hb1), cdt),
                   jax.ShapeDtypeStruct((n, 128), jnp.float32)),
        grid=(n // tm,),
        in_specs=[pl.BlockSpec((tm, din), lambda i: (i, 0)),
                  pl.BlockSpec((din, heads * h_dim), lambda i: (0, 0)),
                  pl.BlockSpec((heads * h_dim, 128), lambda i: (0, 0))],
        out_specs=(pl.BlockSpec((tm, hb1), lambda i: (i, 0)),
                   pl.BlockSpec((tm, 128), lambda i: (i, 0))),
        compiler_params=vlim,
    )(x, w1, acat1)

    ssc1, g1, vg1 = _side_slabs(s1_all, heads)

    attn1 = functools.partial(_attn1_kernel, heads=heads, out_ch=h_dim,
                              out_ch2=z_dim)
    rhs_b, s2_all = pl.pallas_call(
        attn1,
        out_shape=(jax.ShapeDtypeStruct((n, hb2), cdt),
                   jax.ShapeDtypeStruct((n, 128), jnp.float32)),
        grid=(n // tm,),
        in_specs=[
            pl.BlockSpec((tm, 128), lambda i: (i, 0)),
            pl.BlockSpec((8, n), lambda i: (0, 0)),
            pl.BlockSpec((8, n), lambda i: (0, 0)),
            pl.BlockSpec((1, 128), lambda i: (0, 0)),
            pl.BlockSpec((n, hb1), lambda i: (0, 0)),
            pl.BlockSpec((tm, n), lambda i: (i, 0)),
            pl.BlockSpec((1, h_dim), lambda i: (0, 0)),
            pl.BlockSpec((1, h_dim), lambda i: (0, 0)),
            pl.BlockSpec((h_dim, heads * z_dim), lambda i: (0, 0)),
            pl.BlockSpec((heads * z_dim, 128), lambda i: (0, 0)),
        ],
        out_specs=(pl.BlockSpec((tm, hb2), lambda i: (i, 0)),
                   pl.BlockSpec((tm, 128), lambda i: (i, 0))),
        compiler_params=vlim,
    )(s1_all, ssc1, g1, vg1, rhs_a, adj, scale1, shift1, w2, acat2)

    ssc2, g2, vg2 = _side_slabs(s2_all, heads)

    attn2 = functools.partial(_attn2_kernel, heads=heads, out_ch=z_dim)
    out = pl.pallas_call(
        attn2,
        out_shape=jax.ShapeDtypeStruct((n, dout), jnp.float32),
        grid=(n // tm,),
        in_specs=[
            pl.BlockSpec((tm, 128), lambda i: (i, 0)),
            pl.BlockSpec((8, n), lambda i: (0, 0)),
            pl.BlockSpec((8, n), lambda i: (0, 0)),
            pl.BlockSpec((1, 128), lambda i: (0, 0)),
            pl.BlockSpec((n, hb2), lambda i: (0, 0)),
            pl.BlockSpec((tm, n), lambda i: (i, 0)),
            pl.BlockSpec((1, z_dim), lambda i: (0, 0)),
            pl.BlockSpec((1, z_dim), lambda i: (0, 0)),
            pl.BlockSpec((z_dim, h_dim), lambda i: (0, 0)),
            pl.BlockSpec((1, h_dim), lambda i: (0, 0)),
            pl.BlockSpec((h_dim, dout), lambda i: (0, 0)),
            pl.BlockSpec((1, dout), lambda i: (0, 0)),
            pl.BlockSpec((tm, 1), lambda i: (i, 0)),
        ],
        out_specs=pl.BlockSpec((tm, dout), lambda i: (i, 0)),
        compiler_params=vlim,
    )(s2_all, ssc2, g2, vg2, rhs_b, adj, scale2, shift2, w1d, shift1d, w2d,
      dec2_b, size_factors)

    return out[:n_orig]


# speedup vs baseline: 3.6147x; 1.0939x over previous
import functools
import math

import jax
import jax.numpy as jnp
from jax import lax
from jax.experimental import pallas as pl
from jax.experimental.pallas import tpu as pltpu


_SLOPE = 0.2
_SRC0 = 8
_TM = 512
_HB = 256
_CLO = -80.0
_CHI = 30.0


def _build_rhs(hb, s_all, *, heads, out_ch, dtype):
    tm = hb.shape[0]
    zero = jnp.zeros((tm, _HB - out_ch - 1), dtype)
    blocks = []
    for hh in range(heads):
        v = s_all[:, _SRC0 + hh:_SRC0 + hh + 1]
        e2 = jnp.exp2(jnp.clip(_SLOPE * v, _CLO, _CHI)).astype(dtype)
        h_h = hb[:, hh * out_ch:(hh + 1) * out_ch]
        blocks += [h_h * e2, e2, zero]
    return jnp.concatenate(blocks, axis=1)


def _proj_kernel(x_ref, w_ref, acat_ref, r2_ref, s_ref, *, heads, out_ch):
    h32 = jnp.dot(x_ref[...].astype(w_ref.dtype), w_ref[...],
                  preferred_element_type=jnp.float32)
    hb = h32.astype(r2_ref.dtype)
    s_all = jnp.dot(hb, acat_ref[...], preferred_element_type=jnp.float32)
    s_ref[...] = s_all
    r2_ref[...] = _build_rhs(hb, s_all, heads=heads, out_ch=out_ch,
                             dtype=r2_ref.dtype)


def _attn_body(dsc_ref, ssc_ref, g_ref, vg_ref, r2_ref, adj_ref, scale_ref,
               shift_ref, *, heads, out_ch):
    maskb = adj_ref[...].astype(jnp.bfloat16)
    total = None
    for hh in range(heads):
        u = dsc_ref[:, hh:hh + 1]
        s = u + vg_ref[0:1, hh:hh + 1]
        m = jnp.maximum(s, _SLOPE * s)
        f1 = jnp.exp2(jnp.clip(u - m, _CLO, _CHI)).astype(jnp.bfloat16)
        f2 = jnp.exp2(jnp.clip(_SLOPE * u - m, _CLO, _CHI)).astype(
            jnp.bfloat16)
        nu = (-u).astype(jnp.bfloat16)
        a = f1 * g_ref[hh:hh + 1, :]
        pos = ssc_ref[hh:hh + 1, :] > nu
        p = jnp.where(pos, a, f2) * maskb
        sl = slice(hh * _HB, (hh + 1) * _HB)
        d = jnp.dot(p, r2_ref[:, sl], preferred_element_type=jnp.float32)
        o_h = d[:, :out_ch] * pl.reciprocal(d[:, out_ch:out_ch + 1],
                                            approx=True)
        total = o_h if total is None else total + o_h
    y = total * scale_ref[...] + shift_ref[...]
    return jnp.maximum(y, 0.0)


def _attn1_kernel(dsc_ref, ssc_ref, g_ref, vg_ref, r2_ref, adj_ref, scale_ref,
                  shift_ref, w2_ref, acat2_ref, r2o_ref, s2_ref,
                  *, heads, out_ch, out_ch2):
    h1 = _attn_body(dsc_ref, ssc_ref, g_ref, vg_ref, r2_ref, adj_ref,
                    scale_ref, shift_ref, heads=heads, out_ch=out_ch)
    h1b = h1.astype(w2_ref.dtype)
    h2 = jnp.dot(h1b, w2_ref[...], preferred_element_type=jnp.float32)
    h2b = h2.astype(r2o_ref.dtype)
    s2 = jnp.dot(h2b, acat2_ref[...], preferred_element_type=jnp.float32)
    s2_ref[...] = s2
    r2o_ref[...] = _build_rhs(h2b, s2, heads=heads, out_ch=out_ch2,
                              dtype=r2o_ref.dtype)


def _attn2_kernel(dsc_ref, ssc_ref, g_ref, vg_ref, r2_ref, adj_ref, scale_ref,
                  shift_ref, w1d_ref, shift1d_ref, w2d_ref, b2d_ref, sf_ref,
                  o_ref, *, heads, out_ch):
    z = _attn_body(dsc_ref, ssc_ref, g_ref, vg_ref, r2_ref, adj_ref,
                   scale_ref, shift_ref, heads=heads, out_ch=out_ch)
    zb = z.astype(w1d_ref.dtype)
    y = jnp.dot(zb, w1d_ref[...],
                preferred_element_type=jnp.float32) + shift1d_ref[...]
    y = jnp.maximum(y, 0.0).astype(w2d_ref.dtype)
    o = jnp.dot(y, w2d_ref[...],
                preferred_element_type=jnp.float32) + b2d_ref[...]
    o_ref[...] = jnp.maximum(o, 0.0) * sf_ref[...]


def _bn_fold(gamma, beta, mean, var, eps=1e-5):
    scale = gamma * lax.rsqrt(var + eps)
    return scale, beta - mean * scale


def _build_att_matrix(asrc, adst):
    heads, c = asrc.shape
    log2e = jnp.float32(1.4426950408889634)
    cols = jnp.arange(128)
    rows_head = jnp.arange(heads * c) // c
    dst_flat = adst.reshape(-1) * log2e
    src_flat = asrc.reshape(-1) * log2e
    a = jnp.where(cols[None, :] == rows_head[:, None], dst_flat[:, None], 0.0)
    a = jnp.where(cols[None, :] == (_SRC0 + rows_head[:, None]),
                  src_flat[:, None], a)
    return a


def _side_slabs(s_all, heads):
    ssc = s_all[:, _SRC0:_SRC0 + 8].T
    g = jnp.exp2(jnp.clip((1.0 - _SLOPE) * ssc, _CLO, _CHI)).astype(
        jnp.bfloat16)
    vg = jnp.max(ssc[:heads], axis=1)
    vg = jnp.pad(vg, (0, 128 - heads)).reshape(1, 128)
    return ssc.astype(jnp.bfloat16), g, vg


def _round_up(n, m):
    return ((n + m - 1) // m) * m


def kernel(enc1_w, enc1_asrc, enc1_adst, enc1_b, bn1_g, bn1_b, bn1_m, bn1_v,
           enc2_w, enc2_asrc, enc2_adst, enc2_b, bn2_g, bn2_b, bn2_m, bn2_v,
           dec1_w, dec1_b, bn3_g, bn3_b, bn3_m, bn3_v, dec2_w, dec2_b,
           x, adj, size_factors):
    cdt = jnp.bfloat16
    heads, h_dim = enc1_asrc.shape
    z_dim = enc2_asrc.shape[1]
    n_orig = x.shape[0]
    din = x.shape[1]

    tm = max(128, min(_TM, _round_up(n_orig, 128)))
    n = _round_up(n_orig, tm)
    if n > n_orig:
        pad = n - n_orig
        x = jnp.pad(x, ((0, pad), (0, 0)))
        adj = jnp.pad(adj, ((0, pad), (0, pad)))
        size_factors = jnp.pad(size_factors, ((0, pad), (0, 0)))

    s1, t1 = _bn_fold(bn1_g, bn1_b, bn1_m, bn1_v)
    s2, t2 = _bn_fold(bn2_g, bn2_b, bn2_m, bn2_v)
    s3, t3 = _bn_fold(bn3_g, bn3_b, bn3_m, bn3_v)

    acat1 = _build_att_matrix(enc1_asrc, enc1_adst).astype(cdt)
    acat2 = _build_att_matrix(enc2_asrc, enc2_adst).astype(cdt)
    w1 = enc1_w.astype(cdt)
    w2 = enc2_w.astype(cdt)

    scale1 = s1 / heads
    shift1 = enc1_b * s1 + t1
    scale2 = s2 / heads
    shift2 = enc2_b * s2 + t2
    w1d = (dec1_w * s3).astype(cdt)
    shift1d = dec1_b * s3 + t3
    w2d = dec2_w.astype(cdt)

    hb1 = heads * _HB
    hb2 = heads * _HB
    dout = dec2_w.shape[1]
    vlim = pltpu.CompilerParams(
        dimension_semantics=("parallel",),
        vmem_limit_bytes=56 * 1024 * 1024)

    proj = functools.partial(_proj_kernel, heads=heads, out_ch=h_dim)
    rhs_a, s1_all = pl.pallas_call(
        proj,
        out_shape=(jax.ShapeDtypeStruct((n, hb1), cdt),
                   jax.ShapeDtypeStruct((n, 128), jnp.float32)),
        grid=(n // tm,),
        in_specs=[pl.BlockSpec((tm, din), lambda i: (i, 0)),
                  pl.BlockSpec((din, heads * h_dim), lambda i: (0, 0)),
                  pl.BlockSpec((heads * h_dim, 128), lambda i: (0, 0))],
        out_specs=(pl.BlockSpec((tm, hb1), lambda i: (i, 0)),
                   pl.BlockSpec((tm, 128), lambda i: (i, 0))),
        compiler_params=vlim,
    )(x, w1, acat1)

    ssc1, g1, vg1 = _side_slabs(s1_all, heads)

    attn1 = functools.partial(_attn1_kernel, heads=heads, out_ch=h_dim,
                              out_ch2=z_dim)
    rhs_b, s2_all = pl.pallas_call(
        attn1,
        out_shape=(jax.ShapeDtypeStruct((n, hb2), cdt),
                   jax.ShapeDtypeStruct((n, 128), jnp.float32)),
        grid=(n // tm,),
        in_specs=[
            pl.BlockSpec((tm, 128), lambda i: (i, 0)),
            pl.BlockSpec((8, n), lambda i: (0, 0)),
            pl.BlockSpec((8, n), lambda i: (0, 0)),
            pl.BlockSpec((1, 128), lambda i: (0, 0)),
            pl.BlockSpec((n, hb1), lambda i: (0, 0)),
            pl.BlockSpec((tm, n), lambda i: (i, 0)),
            pl.BlockSpec((1, h_dim), lambda i: (0, 0)),
            pl.BlockSpec((1, h_dim), lambda i: (0, 0)),
            pl.BlockSpec((h_dim, heads * z_dim), lambda i: (0, 0)),
            pl.BlockSpec((heads * z_dim, 128), lambda i: (0, 0)),
        ],
        out_specs=(pl.BlockSpec((tm, hb2), lambda i: (i, 0)),
                   pl.BlockSpec((tm, 128), lambda i: (i, 0))),
        compiler_params=vlim,
    )(s1_all, ssc1, g1, vg1, rhs_a, adj, scale1, shift1, w2, acat2)

    ssc2, g2, vg2 = _side_slabs(s2_all, heads)

    attn2 = functools.partial(_attn2_kernel, heads=heads, out_ch=z_dim)
    out = pl.pallas_call(
        attn2,
        out_shape=jax.ShapeDtypeStruct((n, dout), jnp.float32),
        grid=(n // tm,),
        in_specs=[
            pl.BlockSpec((tm, 128), lambda i: (i, 0)),
            pl.BlockSpec((8, n), lambda i: (0, 0)),
            pl.BlockSpec((8, n), lambda i: (0, 0)),
            pl.BlockSpec((1, 128), lambda i: (0, 0)),
            pl.BlockSpec((n, hb2), lambda i: (0, 0)),
            pl.BlockSpec((tm, n), lambda i: (i, 0)),
            pl.BlockSpec((1, z_dim), lambda i: (0, 0)),
            pl.BlockSpec((1, z_dim), lambda i: (0, 0)),
            pl.BlockSpec((z_dim, h_dim), lambda i: (0, 0)),
            pl.BlockSpec((1, h_dim), lambda i: (0, 0)),
            pl.BlockSpec((h_dim, dout), lambda i: (0, 0)),
            pl.BlockSpec((1, dout), lambda i: (0, 0)),
            pl.BlockSpec((tm, 1), lambda i: (i, 0)),
        ],
        out_specs=pl.BlockSpec((tm, dout), lambda i: (i, 0)),
        compiler_params=vlim,
    )(s2_all, ssc2, g2, vg2, rhs_b, adj, scale2, shift2, w1d, shift1d, w2d,
      dec2_b, size_factors)

    return out[:n_orig]
